# baseline (speedup 1.0000x reference)
import functools

import jax
import jax.numpy as jnp
from jax import lax
from jax.experimental import pallas as pl
from jax.experimental.pallas import tpu as pltpu
from jax.experimental.pallas import tpu_sc as plsc

B = 16384
F = 26
FIELD = 40000
CD = 13
TOTAL_ROWS = F * FIELD

NC = 2
NS = 16
NW = NC * NS
BPW = B // NW
CHUNK = 128
CPF = BPW // CHUNK
NGATHER = F * CPF


def _body(x_hbm, xc_hbm, table_hbm, w_hbm, b_hbm, out_hbm,
          xbuf, cbuf, ibuf, gbuf, obuf, wv, bv, sem):
    wid = lax.axis_index("s") * NC + lax.axis_index("c")
    base = pl.multiple_of(wid * BPW, BPW)

    pltpu.sync_copy(x_hbm.at[pl.ds(pl.multiple_of(wid * (BPW * F), 8), BPW * F)], xbuf)
    pltpu.sync_copy(xc_hbm.at[pl.ds(pl.multiple_of(wid * (BPW * CD), 8), BPW * CD)], cbuf)
    pltpu.sync_copy(w_hbm, wv)
    pltpu.sync_copy(b_hbm, bv)

    lane = lax.iota(jnp.int32, 16)
    lane_f = lane * F
    lane_cd = lane * CD

    @pl.loop(0, CPF)
    def _transpose(cc):
        for c8 in range(8):
            rowoff = (cc * 8 + c8) * (16 * F)
            for j in range(F):
                g = plsc.load_gather(xbuf, [lane_f + (rowoff + j)])
                ibuf[j * CPF + cc, pl.ds(c8 * 16, 16)] = g + j * FIELD

    @pl.loop(0, NGATHER // 8)
    def _gather(blk):
        copies = []
        for i in range(8):
            k = blk * 8 + i
            copies.append(
                pltpu.async_copy(table_hbm.at[ibuf.at[k]], gbuf.at[k], sem))
        for c in copies:
            c.wait()

    bias_vec = bv[...]
    wsp = [wv[jj, :] for jj in range(CD)]

    @pl.loop(0, CPF)
    def _reduce(cc):
        for c8 in range(8):
            c = cc * 8 + c8
            acc = bias_vec
            cro = c * (16 * CD)
            for jj in range(CD):
                vals = plsc.load_gather(cbuf, [lane_cd + (cro + jj)])
                acc = acc + vals * wsp[jj]
            for j in range(F):
                acc = acc + gbuf[j * CPF + cc, pl.ds(c8 * 16, 16)]
            obuf[pl.ds(pl.multiple_of(c * 16, 16), 16)] = acc

    pltpu.sync_copy(obuf, out_hbm.at[pl.ds(base, BPW)])


_sc_call = pl.kernel(
    _body,
    out_type=jax.ShapeDtypeStruct((B,), jnp.float32),
    mesh=plsc.VectorSubcoreMesh(core_axis_name="c", subcore_axis_name="s",
                                num_cores=NC, num_subcores=NS),
    compiler_params=pltpu.CompilerParams(needs_layout_passes=False),
    scratch_types=[
        pltpu.VMEM((BPW * F,), jnp.int32),
        pltpu.VMEM((BPW * CD,), jnp.float32),
        pltpu.VMEM((NGATHER, CHUNK), jnp.int32),
        pltpu.VMEM((NGATHER, CHUNK), jnp.float32),
        pltpu.VMEM((BPW,), jnp.float32),
        pltpu.VMEM((CD, 16), jnp.float32),
        pltpu.VMEM((16,), jnp.float32),
        pltpu.SemaphoreType.DMA,
    ],
)


@jax.jit
def kernel(x, x_cont, emb_x, table, bias, w):
    del emb_x
    xf = x.reshape(-1).astype(jnp.int32)
    cf = x_cont.reshape(-1)
    tf = table.reshape(-1)
    wb = jnp.broadcast_to(w[:, None], (CD, 16))
    b16 = jnp.broadcast_to(bias, (16,))
    out = _sc_call(xf, cf, tf, wb, b16)
    return out.reshape(B, 1)


# speedup vs baseline: 1.1199x; 1.1199x over previous
import jax
import jax.numpy as jnp
from jax import lax
from jax.experimental import pallas as pl
from jax.experimental.pallas import tpu as pltpu
from jax.experimental.pallas import tpu_sc as plsc

B = 16384
F = 26
FIELD = 40000
CD = 13

NC = 2
NS = 16
NW = NC * NS
BPW = B // NW
CHUNK = 128
NBLK = BPW // CHUNK
NGATHER = F * NBLK


def _body(x_hbm, xc_hbm, table_hbm, w_hbm, b_hbm, out_hbm,
          xbuf, cbuf, ibuf, gbuf, obuf, wv, bv, sem):
    wid = lax.axis_index("s") * NC + lax.axis_index("c")
    base = pl.multiple_of(wid * BPW, BPW)

    pltpu.sync_copy(x_hbm.at[pl.ds(pl.multiple_of(wid * (BPW * F), 8), BPW * F)], xbuf)
    pltpu.sync_copy(xc_hbm.at[pl.ds(pl.multiple_of(wid * (BPW * CD), 8), BPW * CD)], cbuf)
    pltpu.sync_copy(w_hbm, wv)
    pltpu.sync_copy(b_hbm, bv)

    lane = lax.iota(jnp.int32, 16)
    lane_f = lane * F
    lane_cd = lane * CD

    @pl.loop(0, NBLK)
    def _build_and_fire(cc):
        for j in range(F):
            row = cc * F + j
            for c8 in range(8):
                rowoff = (cc * 8 + c8) * (16 * F)
                g = plsc.load_gather(xbuf, [lane_f + (rowoff + j)])
                ibuf[row, pl.ds(c8 * 16, 16)] = g + j * FIELD
            pltpu.async_copy(table_hbm.at[ibuf.at[row]], gbuf.at[row], sem)

    bias_vec = bv[...]
    wsp = [wv[jj, :] for jj in range(CD)]

    @pl.loop(0, NBLK)
    def _reduce(cc):
        for j in range(F):
            row = cc * F + j
            pltpu.make_async_copy(table_hbm.at[ibuf.at[row]], gbuf.at[row], sem).wait()
        for c8 in range(8):
            c = cc * 8 + c8
            acc = bias_vec
            cro = c * (16 * CD)
            for jj in range(CD):
                vals = plsc.load_gather(cbuf, [lane_cd + (cro + jj)])
                acc = acc + vals * wsp[jj]
            for j in range(F):
                acc = acc + gbuf[cc * F + j, pl.ds(c8 * 16, 16)]
            obuf[pl.ds(pl.multiple_of(c * 16, 16), 16)] = acc

    pltpu.sync_copy(obuf, out_hbm.at[pl.ds(base, BPW)])


_sc_call = pl.kernel(
    _body,
    out_type=jax.ShapeDtypeStruct((B,), jnp.float32),
    mesh=plsc.VectorSubcoreMesh(core_axis_name="c", subcore_axis_name="s",
                                num_cores=NC, num_subcores=NS),
    compiler_params=pltpu.CompilerParams(needs_layout_passes=False),
    scratch_types=[
        pltpu.VMEM((BPW * F,), jnp.int32),
        pltpu.VMEM((BPW * CD,), jnp.float32),
        pltpu.VMEM((NGATHER, CHUNK), jnp.int32),
        pltpu.VMEM((NGATHER, CHUNK), jnp.float32),
        pltpu.VMEM((BPW,), jnp.float32),
        pltpu.VMEM((CD, 16), jnp.float32),
        pltpu.VMEM((16,), jnp.float32),
        pltpu.SemaphoreType.DMA,
    ],
)


@jax.jit
def kernel(x, x_cont, emb_x, table, bias, w):
    del emb_x
    xf = x.reshape(-1).astype(jnp.int32)
    cf = x_cont.reshape(-1)
    tf = table.reshape(-1)
    wb = jnp.broadcast_to(w[:, None], (CD, 16))
    b16 = jnp.broadcast_to(bias, (16,))
    out = _sc_call(xf, cf, tf, wb, b16)
    return out.reshape(B, 1)


# speedup vs baseline: 1.2031x; 1.0743x over previous
import jax
import jax.numpy as jnp
from jax import lax
from jax.experimental import pallas as pl
from jax.experimental.pallas import tpu as pltpu
from jax.experimental.pallas import tpu_sc as plsc

B = 16384
F = 26
FIELD = 40000
CD = 13
TROWS = F * FIELD

NC = 2
NS = 16
NW = NC * NS
BPW = B // NW
CHUNK = 128
NBLK = BPW // CHUNK
C16 = CHUNK // 16
NGATHER = F * NBLK
TSH0 = 65024
TSHL = TROWS - 15 * TSH0


def _body(x_hbm, xc_hbm, table_hbm, w_hbm, b_hbm, out_hbm,
          xbuf, cbuf, ibuf, gbuf, obuf, wv, bv, tsh, sem, sem2):
    sid = lax.axis_index("s")
    wid = sid * NC + lax.axis_index("c")
    base = pl.multiple_of(wid * BPW, BPW)

    toff = pl.multiple_of(sid * TSH0, 128)

    @pl.when(sid < NS - 1)
    def _():
        pltpu.async_copy(table_hbm.at[pl.ds(toff, TSH0)],
                         tsh.at[pl.ds(toff, TSH0)], sem2)

    @pl.when(sid == NS - 1)
    def _():
        pltpu.async_copy(table_hbm.at[pl.ds(15 * TSH0, TSHL)],
                         tsh.at[pl.ds(15 * TSH0, TSHL)], sem2)

    pltpu.sync_copy(x_hbm.at[pl.ds(pl.multiple_of(wid * (BPW * F), 8), BPW * F)], xbuf)
    pltpu.sync_copy(xc_hbm.at[pl.ds(pl.multiple_of(wid * (BPW * CD), 8), BPW * CD)], cbuf)
    pltpu.sync_copy(w_hbm, wv)
    pltpu.sync_copy(b_hbm, bv)

    lane = lax.iota(jnp.int32, 16)
    lane_f = lane * F
    lane_cd = lane * CD

    @pl.loop(0, NBLK)
    def _build(cc):
        for j in range(F):
            row = cc * F + j
            for c8 in range(C16):
                rowoff = (cc * C16 + c8) * (16 * F)
                g = plsc.load_gather(xbuf, [lane_f + (rowoff + j)])
                ibuf[row, pl.ds(c8 * 16, 16)] = g + j * FIELD

    @pl.when(sid < NS - 1)
    def _():
        pltpu.make_async_copy(table_hbm.at[pl.ds(toff, TSH0)],
                              tsh.at[pl.ds(toff, TSH0)], sem2).wait()

    @pl.when(sid == NS - 1)
    def _():
        pltpu.make_async_copy(table_hbm.at[pl.ds(15 * TSH0, TSHL)],
                              tsh.at[pl.ds(15 * TSH0, TSHL)], sem2).wait()

    plsc.subcore_barrier()

    @pl.loop(0, NBLK)
    def _fire(cc):
        for j in range(F):
            row = cc * F + j
            pltpu.async_copy(tsh.at[ibuf.at[row]], gbuf.at[row], sem)

    bias_vec = bv[...]
    wsp = [wv[jj, :] for jj in range(CD)]

    @pl.loop(0, NBLK)
    def _reduce(cc):
        for j in range(F):
            row = cc * F + j
            pltpu.make_async_copy(tsh.at[ibuf.at[row]], gbuf.at[row], sem).wait()
        for c8 in range(C16):
            c = cc * C16 + c8
            acc = bias_vec
            cro = c * (16 * CD)
            for jj in range(CD):
                vals = plsc.load_gather(cbuf, [lane_cd + (cro + jj)])
                acc = acc + vals * wsp[jj]
            for j in range(F):
                acc = acc + gbuf[cc * F + j, pl.ds(c8 * 16, 16)]
            obuf[pl.ds(pl.multiple_of(c * 16, 16), 16)] = acc

    pltpu.sync_copy(obuf, out_hbm.at[pl.ds(base, BPW)])


_sc_call = pl.kernel(
    _body,
    out_type=jax.ShapeDtypeStruct((B,), jnp.float32),
    mesh=plsc.VectorSubcoreMesh(core_axis_name="c", subcore_axis_name="s",
                                num_cores=NC, num_subcores=NS),
    compiler_params=pltpu.CompilerParams(needs_layout_passes=False),
    scratch_types=[
        pltpu.VMEM((BPW * F,), jnp.int32),
        pltpu.VMEM((BPW * CD,), jnp.float32),
        pltpu.VMEM((NGATHER, CHUNK), jnp.int32),
        pltpu.VMEM((NGATHER, CHUNK), jnp.float32),
        pltpu.VMEM((BPW,), jnp.float32),
        pltpu.VMEM((CD, 16), jnp.float32),
        pltpu.VMEM((16,), jnp.float32),
        pltpu.VMEM_SHARED((TROWS,), jnp.float32),
        pltpu.SemaphoreType.DMA,
        pltpu.SemaphoreType.DMA,
    ],
)


@jax.jit
def kernel(x, x_cont, emb_x, table, bias, w):
    del emb_x
    xf = x.reshape(-1).astype(jnp.int32)
    cf = x_cont.reshape(-1)
    tf = table.reshape(-1)
    wb = jnp.broadcast_to(w[:, None], (CD, 16))
    b16 = jnp.broadcast_to(bias, (16,))
    out = _sc_call(xf, cf, tf, wb, b16)
    return out.reshape(B, 1)


# speedup vs baseline: 1.2102x; 1.0059x over previous
import jax
import jax.numpy as jnp
from jax import lax
from jax.experimental import pallas as pl
from jax.experimental.pallas import tpu as pltpu
from jax.experimental.pallas import tpu_sc as plsc

B = 16384
F = 26
FIELD = 40000
CD = 13
TROWS = F * FIELD

NC = 2
NS = 16
NW = NC * NS
BPW = B // NW
CHUNK = 128
NBLK = BPW // CHUNK
C16 = CHUNK // 16
NGATHER = F * NBLK
TSH0 = 65024
TSHL = TROWS - 15 * TSH0


def _body(x_hbm, xc_hbm, table_hbm, w_hbm, b_hbm, out_hbm,
          xbuf, cbuf, ibuf, gbuf, obuf, wv, bv, tsh, sem, sem2, semx, semi):
    sid = lax.axis_index("s")
    wid = sid * NC + lax.axis_index("c")
    base = pl.multiple_of(wid * BPW, BPW)

    toff = pl.multiple_of(sid * TSH0, 128)

    @pl.when(sid < NS - 1)
    def _():
        pltpu.async_copy(table_hbm.at[pl.ds(toff, TSH0)],
                         tsh.at[pl.ds(toff, TSH0)], sem2)

    @pl.when(sid == NS - 1)
    def _():
        pltpu.async_copy(table_hbm.at[pl.ds(15 * TSH0, TSHL)],
                         tsh.at[pl.ds(15 * TSH0, TSHL)], sem2)

    cpx = pltpu.async_copy(
        x_hbm.at[pl.ds(pl.multiple_of(wid * (BPW * F), 8), BPW * F)], xbuf, semx)
    cpc = pltpu.async_copy(
        xc_hbm.at[pl.ds(pl.multiple_of(wid * (BPW * CD), 8), BPW * CD)], cbuf, semi)
    cpw = pltpu.async_copy(w_hbm, wv, semi)
    cpb = pltpu.async_copy(b_hbm, bv, semi)
    cpx.wait()

    lane = lax.iota(jnp.int32, 16)
    lane_f = lane * F
    lane_cd = lane * CD

    @pl.loop(0, NBLK)
    def _build(cc):
        for j in range(F):
            row = cc * F + j
            for c8 in range(C16):
                rowoff = (cc * C16 + c8) * (16 * F)
                g = plsc.load_gather(xbuf, [lane_f + (rowoff + j)])
                ibuf[row, pl.ds(c8 * 16, 16)] = g + j * FIELD

    @pl.when(sid < NS - 1)
    def _():
        pltpu.make_async_copy(table_hbm.at[pl.ds(toff, TSH0)],
                              tsh.at[pl.ds(toff, TSH0)], sem2).wait()

    @pl.when(sid == NS - 1)
    def _():
        pltpu.make_async_copy(table_hbm.at[pl.ds(15 * TSH0, TSHL)],
                              tsh.at[pl.ds(15 * TSH0, TSHL)], sem2).wait()

    plsc.subcore_barrier()

    @pl.loop(0, NBLK)
    def _fire(cc):
        for j in range(F):
            row = cc * F + j
            pltpu.async_copy(tsh.at[ibuf.at[row]], gbuf.at[row], sem)

    cpc.wait()
    cpw.wait()
    cpb.wait()
    bias_vec = bv[...]
    wsp = [wv[jj, :] for jj in range(CD)]

    @pl.loop(0, NBLK)
    def _cont(cc):
        for c8 in range(C16):
            c = cc * C16 + c8
            acc = bias_vec
            cro = c * (16 * CD)
            for jj in range(CD):
                vals = plsc.load_gather(cbuf, [lane_cd + (cro + jj)])
                acc = acc + vals * wsp[jj]
            obuf[pl.ds(pl.multiple_of(c * 16, 16), 16)] = acc

    @pl.loop(0, NBLK)
    def _reduce(cc):
        for j in range(F):
            row = cc * F + j
            pltpu.make_async_copy(tsh.at[ibuf.at[row]], gbuf.at[row], sem).wait()
        for c8 in range(C16):
            c = cc * C16 + c8
            off = pl.multiple_of(c * 16, 16)
            acc = obuf[pl.ds(off, 16)]
            for j in range(F):
                acc = acc + gbuf[cc * F + j, pl.ds(c8 * 16, 16)]
            obuf[pl.ds(off, 16)] = acc

    pltpu.sync_copy(obuf, out_hbm.at[pl.ds(base, BPW)])


_sc_call = pl.kernel(
    _body,
    out_type=jax.ShapeDtypeStruct((B,), jnp.float32),
    mesh=plsc.VectorSubcoreMesh(core_axis_name="c", subcore_axis_name="s",
                                num_cores=NC, num_subcores=NS),
    compiler_params=pltpu.CompilerParams(needs_layout_passes=False),
    scratch_types=[
        pltpu.VMEM((BPW * F,), jnp.int32),
        pltpu.VMEM((BPW * CD,), jnp.float32),
        pltpu.VMEM((NGATHER, CHUNK), jnp.int32),
        pltpu.VMEM((NGATHER, CHUNK), jnp.float32),
        pltpu.VMEM((BPW,), jnp.float32),
        pltpu.VMEM((CD, 16), jnp.float32),
        pltpu.VMEM((16,), jnp.float32),
        pltpu.VMEM_SHARED((TROWS,), jnp.float32),
        pltpu.SemaphoreType.DMA,
        pltpu.SemaphoreType.DMA,
        pltpu.SemaphoreType.DMA,
        pltpu.SemaphoreType.DMA,
    ],
)


@jax.jit
def kernel(x, x_cont, emb_x, table, bias, w):
    del emb_x
    xf = x.reshape(-1).astype(jnp.int32)
    cf = x_cont.reshape(-1)
    tf = table.reshape(-1)
    wb = jnp.broadcast_to(w[:, None], (CD, 16))
    b16 = jnp.broadcast_to(bias, (16,))
    out = _sc_call(xf, cf, tf, wb, b16)
    return out.reshape(B, 1)


# speedup vs baseline: 1.2118x; 1.0013x over previous
import jax
import jax.numpy as jnp
from jax import lax
from jax.experimental import pallas as pl
from jax.experimental.pallas import tpu as pltpu
from jax.experimental.pallas import tpu_sc as plsc

B = 16384
F = 26
FIELD = 40000
CD = 13
TROWS = F * FIELD

NC = 2
NS = 16
NW = NC * NS
BPW = B // NW
CHUNK = 128
NBLK = BPW // CHUNK
C16 = CHUNK // 16
NGATHER = F * NBLK
HF = 10
TSH0 = 65024
TSHL = TROWS - 15 * TSH0


def _body(x_hbm, xc_hbm, table_hbm, w_hbm, b_hbm, out_hbm,
          xbuf, cbuf, ibuf, gbuf, obuf, wv, bv, tsh, sem, sem2, semx, semi, semh):
    sid = lax.axis_index("s")
    wid = sid * NC + lax.axis_index("c")
    base = pl.multiple_of(wid * BPW, BPW)

    toff = pl.multiple_of(sid * TSH0, 128)

    @pl.when(sid < NS - 1)
    def _():
        pltpu.async_copy(table_hbm.at[pl.ds(toff, TSH0)],
                         tsh.at[pl.ds(toff, TSH0)], sem2)

    @pl.when(sid == NS - 1)
    def _():
        pltpu.async_copy(table_hbm.at[pl.ds(15 * TSH0, TSHL)],
                         tsh.at[pl.ds(15 * TSH0, TSHL)], sem2)

    cpx = pltpu.async_copy(
        x_hbm.at[pl.ds(pl.multiple_of(wid * (BPW * F), 8), BPW * F)], xbuf, semx)
    cpc = pltpu.async_copy(
        xc_hbm.at[pl.ds(pl.multiple_of(wid * (BPW * CD), 8), BPW * CD)], cbuf, semi)
    cpw = pltpu.async_copy(w_hbm, wv, semi)
    cpb = pltpu.async_copy(b_hbm, bv, semi)
    cpx.wait()

    lane = lax.iota(jnp.int32, 16)
    lane_f = lane * F
    lane_cd = lane * CD

    @pl.loop(0, NBLK)
    def _build(cc):
        for j in range(F):
            row = cc * F + j
            for c8 in range(C16):
                rowoff = (cc * C16 + c8) * (16 * F)
                g = plsc.load_gather(xbuf, [lane_f + (rowoff + j)])
                ibuf[row, pl.ds(c8 * 16, 16)] = g + j * FIELD
        for j in range(HF):
            row = cc * F + j
            pltpu.async_copy(table_hbm.at[ibuf.at[row]], gbuf.at[row], semh)

    @pl.when(sid < NS - 1)
    def _():
        pltpu.make_async_copy(table_hbm.at[pl.ds(toff, TSH0)],
                              tsh.at[pl.ds(toff, TSH0)], sem2).wait()

    @pl.when(sid == NS - 1)
    def _():
        pltpu.make_async_copy(table_hbm.at[pl.ds(15 * TSH0, TSHL)],
                              tsh.at[pl.ds(15 * TSH0, TSHL)], sem2).wait()

    plsc.subcore_barrier()

    @pl.loop(0, NBLK)
    def _fire(cc):
        for j in range(HF, F):
            row = cc * F + j
            pltpu.async_copy(tsh.at[ibuf.at[row]], gbuf.at[row], sem)

    cpc.wait()
    cpw.wait()
    cpb.wait()
    bias_vec = bv[...]
    wsp = [wv[jj, :] for jj in range(CD)]

    @pl.loop(0, NBLK)
    def _cont(cc):
        for c8 in range(C16):
            c = cc * C16 + c8
            acc = bias_vec
            cro = c * (16 * CD)
            for jj in range(CD):
                vals = plsc.load_gather(cbuf, [lane_cd + (cro + jj)])
                acc = acc + vals * wsp[jj]
            obuf[pl.ds(pl.multiple_of(c * 16, 16), 16)] = acc

    @pl.loop(0, NBLK)
    def _reduce(cc):
        for j in range(HF):
            row = cc * F + j
            pltpu.make_async_copy(table_hbm.at[ibuf.at[row]], gbuf.at[row], semh).wait()
        for j in range(HF, F):
            row = cc * F + j
            pltpu.make_async_copy(tsh.at[ibuf.at[row]], gbuf.at[row], sem).wait()
        for c8 in range(C16):
            c = cc * C16 + c8
            off = pl.multiple_of(c * 16, 16)
            acc = obuf[pl.ds(off, 16)]
            for j in range(F):
                acc = acc + gbuf[cc * F + j, pl.ds(c8 * 16, 16)]
            obuf[pl.ds(off, 16)] = acc

    pltpu.sync_copy(obuf, out_hbm.at[pl.ds(base, BPW)])


_sc_call = pl.kernel(
    _body,
    out_type=jax.ShapeDtypeStruct((B,), jnp.float32),
    mesh=plsc.VectorSubcoreMesh(core_axis_name="c", subcore_axis_name="s",
                                num_cores=NC, num_subcores=NS),
    compiler_params=pltpu.CompilerParams(needs_layout_passes=False),
    scratch_types=[
        pltpu.VMEM((BPW * F,), jnp.int32),
        pltpu.VMEM((BPW * CD,), jnp.float32),
        pltpu.VMEM((NGATHER, CHUNK), jnp.int32),
        pltpu.VMEM((NGATHER, CHUNK), jnp.float32),
        pltpu.VMEM((BPW,), jnp.float32),
        pltpu.VMEM((CD, 16), jnp.float32),
        pltpu.VMEM((16,), jnp.float32),
        pltpu.VMEM_SHARED((TROWS,), jnp.float32),
        pltpu.SemaphoreType.DMA,
        pltpu.SemaphoreType.DMA,
        pltpu.SemaphoreType.DMA,
        pltpu.SemaphoreType.DMA,
        pltpu.SemaphoreType.DMA,
    ],
)


@jax.jit
def kernel(x, x_cont, emb_x, table, bias, w):
    del emb_x
    xf = x.reshape(-1).astype(jnp.int32)
    cf = x_cont.reshape(-1)
    tf = table.reshape(-1)
    wb = jnp.broadcast_to(w[:, None], (CD, 16))
    b16 = jnp.broadcast_to(bias, (16,))
    out = _sc_call(xf, cf, tf, wb, b16)
    return out.reshape(B, 1)
